# Initial kernel scaffold; baseline (speedup 1.0000x reference)
import jax
import jax.numpy as jnp
from jax.experimental import pallas as pl


def kernel(self_feats, neigh_feats, weight):
    raise NotImplementedError("write your pallas kernel here")


# speedup vs baseline: 2.1567x; 2.1567x over previous
import jax
import jax.numpy as jnp
from jax.experimental import pallas as pl

_THRESHOLDS = (0.5, 0.5, 0.5)


def _body(s_ref, n_ref, w_ref, o_ref):
    agg = s_ref[...]
    for r, t in enumerate(_THRESHOLDS):
        agg = agg + t * n_ref[r]
    o_ref[...] = jnp.maximum(
        jnp.dot(agg, w_ref[...], preferred_element_type=jnp.float32), 0.0
    )


def kernel(self_feats, neigh_feats, weight):
    n, f = self_feats.shape
    e = weight.shape[1]
    nrel = neigh_feats.shape[0] // n
    block = 2000
    assert n % block == 0
    neigh3 = neigh_feats.reshape(nrel, n, f)
    return pl.pallas_call(
        _body,
        grid=(n // block,),
        in_specs=[
            pl.BlockSpec((block, f), lambda i: (i, 0)),
            pl.BlockSpec((nrel, block, f), lambda i: (0, i, 0)),
            pl.BlockSpec((f, e), lambda i: (0, 0)),
        ],
        out_specs=pl.BlockSpec((block, e), lambda i: (i, 0)),
        out_shape=jax.ShapeDtypeStruct((n, e), jnp.float32),
    )(self_feats, neigh3, weight)


# speedup vs baseline: 2.2635x; 1.0495x over previous
import jax
import jax.numpy as jnp
from jax.experimental import pallas as pl
from jax.experimental.pallas import tpu as pltpu

_THRESHOLDS = (0.5, 0.5, 0.5)


def _body(s_ref, n_ref, w_ref, o_ref):
    agg = s_ref[...]
    for r, t in enumerate(_THRESHOLDS):
        agg = agg + t * n_ref[r]
    o_ref[...] = jnp.maximum(
        jnp.dot(agg, w_ref[...], preferred_element_type=jnp.float32), 0.0
    )


def kernel(self_feats, neigh_feats, weight):
    n, f = self_feats.shape
    e = weight.shape[1]
    nrel = neigh_feats.shape[0] // n
    block = 5000
    assert n % block == 0
    neigh3 = neigh_feats.reshape(nrel, n, f)
    return pl.pallas_call(
        _body,
        grid=(n // block,),
        in_specs=[
            pl.BlockSpec((block, f), lambda i: (i, 0)),
            pl.BlockSpec((nrel, block, f), lambda i: (0, i, 0)),
            pl.BlockSpec((f, e), lambda i: (0, 0)),
        ],
        out_specs=pl.BlockSpec((block, e), lambda i: (i, 0)),
        out_shape=jax.ShapeDtypeStruct((n, e), jnp.float32),
        compiler_params=pltpu.CompilerParams(
            dimension_semantics=("arbitrary",),
        ),
    )(self_feats, neigh3, weight)
